# baseline (speedup 1.0000x reference)
import functools

import jax
import jax.numpy as jnp
from jax import lax
from jax.experimental import pallas as pl
from jax.experimental.pallas import tpu as pltpu
from jax.experimental.pallas import tpu_sc as plsc

N_DOWN = 10000
N_UP = 10000
E = 320000
D = 128

NC = 2
NS = 16
NW = NC * NS

CHUNK = 64
CPW_A = 80
CPW_B = 240
DEPTH = 5
EPW_A = CPW_A * CHUNK
EPW_B = CPW_B * CHUNK
E_PAD = (EPW_A + EPW_B) * NS
P_ROWS = 10240
ZROWS_PT = P_ROWS // NS
OROWS_PT = 624
OTAIL = N_UP - OROWS_PT * NS

_mesh = plsc.VectorSubcoreMesh(core_axis_name="c", subcore_axis_name="s")


@functools.partial(
    pl.kernel,
    mesh=_mesh,
    out_type=jax.ShapeDtypeStruct((NC, N_UP, D), jnp.float32),
    scratch_types=[
        pltpu.VMEM_SHARED((P_ROWS, D), jnp.float32),
        pltpu.VMEM((DEPTH, CHUNK), jnp.int32),
        pltpu.VMEM((DEPTH, CHUNK), jnp.int32),
        pltpu.VMEM((DEPTH, CHUNK, D), jnp.float32),
        pltpu.VMEM((16, D), jnp.float32),
    ] + [pltpu.SemaphoreType.DMA] * (2 * DEPTH),
)
def _sc_agg(down_hbm, src_hbm, dst_hbm, out_hbm,
            acc, src_v, dst_v, rows_v, zbuf, *sems):
    c = lax.axis_index("c")
    s = lax.axis_index("s")
    sem_r = sems[:DEPTH]
    sem_i = sems[DEPTH:]
    base = jnp.where(c == 0, s * EPW_A, NS * EPW_A + s * EPW_B)
    nch = jnp.where(c == 0, CPW_A, CPW_B)

    for i in range(16):
        for j in range(D // 16):
            zbuf[i, pl.ds(j * 16, 16)] = jnp.zeros((16,), jnp.float32)

    def zero_body(k, carry):
        pltpu.sync_copy(zbuf, acc.at[pl.ds(s * ZROWS_PT + k * 16, 16)])
        return carry

    lax.fori_loop(0, ZROWS_PT // 16, zero_body, 0)
    plsc.subcore_barrier()

    def idx_start(k, b):
        pltpu.async_copy(src_hbm.at[pl.ds(base + k * CHUNK, CHUNK)],
                         src_v.at[b], sem_i[b])
        pltpu.async_copy(dst_hbm.at[pl.ds(base + k * CHUNK, CHUNK)],
                         dst_v.at[b], sem_i[b])

    def idx_wait(k, b):
        pltpu.make_async_copy(src_hbm.at[pl.ds(base + k * CHUNK, CHUNK)],
                              src_v.at[b], sem_i[b]).wait()
        pltpu.make_async_copy(dst_hbm.at[pl.ds(base + k * CHUNK, CHUNK)],
                              dst_v.at[b], sem_i[b]).wait()

    def gather_start(b):
        pltpu.async_copy(down_hbm.at[src_v.at[b]], rows_v.at[b], sem_r[b])

    def gather_wait(b):
        pltpu.make_async_copy(down_hbm.at[src_v.at[b]],
                              rows_v.at[b], sem_r[b]).wait()

    for j in range(DEPTH):
        idx_start(j, j)
    for j in range(DEPTH - 1):
        idx_wait(j, j)
        gather_start(j)

    def body(m, carry):
        for b0 in range(DEPTH):
            k = m * DEPTH + b0
            b = b0
            bn = (b0 + DEPTH - 1) % DEPTH

            @pl.when(k + DEPTH - 1 < nch)
            def _():
                idx_wait(k + DEPTH - 1, bn)
                gather_start(bn)

            gather_wait(b)
            pltpu.sync_copy(rows_v.at[b], acc.at[dst_v.at[b]], add=True)

            @pl.when(k + DEPTH < nch)
            def _():
                idx_start(k + DEPTH, b)
        return carry

    lax.fori_loop(0, nch // DEPTH, body, 0)
    plsc.subcore_barrier()

    pltpu.sync_copy(acc.at[pl.ds(s * OROWS_PT, OROWS_PT)],
                    out_hbm.at[c, pl.ds(s * OROWS_PT, OROWS_PT)])

    @pl.when(s == 0)
    def _():
        pltpu.sync_copy(acc.at[pl.ds(OROWS_PT * NS, OTAIL)],
                        out_hbm.at[c, pl.ds(OROWS_PT * NS, OTAIL)])


def _add_body(a_ref, b_ref, o_ref):
    o_ref[...] = a_ref[...] + b_ref[...]


_BLK = 1000


def _combine(p0, p1):
    return pl.pallas_call(
        _add_body,
        grid=(N_UP // _BLK,),
        in_specs=[pl.BlockSpec((_BLK, D), lambda i: (i, 0))] * 2,
        out_specs=pl.BlockSpec((_BLK, D), lambda i: (i, 0)),
        out_shape=jax.ShapeDtypeStruct((N_UP, D), jnp.float32),
    )(p0, p1)


def kernel(down_nf, edge_index):
    src = edge_index[0]
    dst = edge_index[1]
    pad = E_PAD - E
    src_p = jnp.concatenate([src, jnp.zeros((pad,), jnp.int32)])
    dst_p = jnp.concatenate([dst, jnp.full((pad,), N_UP, jnp.int32)])
    partials = _sc_agg(down_nf, src_p, dst_p)
    return _combine(partials[0], partials[1])


# speedup vs baseline: 1.0837x; 1.0837x over previous
import functools

import jax
import jax.numpy as jnp
from jax import lax
from jax.experimental import pallas as pl
from jax.experimental.pallas import tpu as pltpu
from jax.experimental.pallas import tpu_sc as plsc

N_DOWN = 10000
N_UP = 10000
E = 320000
D = 128

NC = 2
NS = 16
NW = NC * NS

CHUNK = 64
CPW = 160
DEPTH = 5
EPW = CPW * CHUNK
E_PAD = EPW * NW
P_ROWS = 10240
ZROWS_PT = P_ROWS // NS
OROWS_PT = 624
OTAIL = N_UP - OROWS_PT * NS

_mesh = plsc.VectorSubcoreMesh(core_axis_name="c", subcore_axis_name="s")


@functools.partial(
    pl.kernel,
    mesh=_mesh,
    out_type=jax.ShapeDtypeStruct((NC, N_UP, D), jnp.float32),
    scratch_types=[
        pltpu.VMEM_SHARED((P_ROWS, D), jnp.float32),
        pltpu.VMEM((DEPTH, CHUNK), jnp.int32),
        pltpu.VMEM((DEPTH, CHUNK), jnp.int32),
        pltpu.VMEM((DEPTH, CHUNK, D), jnp.float32),
        pltpu.VMEM((16, D), jnp.float32),
    ] + [pltpu.SemaphoreType.DMA] * (2 * DEPTH),
)
def _sc_agg(down_hbm, src_hbm, dst_hbm, out_hbm,
            acc, src_v, dst_v, rows_v, zbuf, *sems):
    c = lax.axis_index("c")
    s = lax.axis_index("s")
    wid = c * NS + s
    sem_r = sems[:DEPTH]
    sem_i = sems[DEPTH:]
    base = wid * EPW

    for i in range(16):
        for j in range(D // 16):
            zbuf[i, pl.ds(j * 16, 16)] = jnp.zeros((16,), jnp.float32)

    def zero_body(k, carry):
        pltpu.sync_copy(zbuf, acc.at[pl.ds(s * ZROWS_PT + k * 16, 16)])
        return carry

    lax.fori_loop(0, ZROWS_PT // 16, zero_body, 0)
    plsc.subcore_barrier()

    def idx_start(k, b):
        pltpu.async_copy(src_hbm.at[pl.ds(base + k * CHUNK, CHUNK)],
                         src_v.at[b], sem_i[b])
        pltpu.async_copy(dst_hbm.at[pl.ds(base + k * CHUNK, CHUNK)],
                         dst_v.at[b], sem_i[b])

    def idx_wait(k, b):
        pltpu.make_async_copy(src_hbm.at[pl.ds(base + k * CHUNK, CHUNK)],
                              src_v.at[b], sem_i[b]).wait()
        pltpu.make_async_copy(dst_hbm.at[pl.ds(base + k * CHUNK, CHUNK)],
                              dst_v.at[b], sem_i[b]).wait()

    def gather_start(b):
        pltpu.async_copy(down_hbm.at[src_v.at[b]], rows_v.at[b], sem_r[b])

    def gather_wait(b):
        pltpu.make_async_copy(down_hbm.at[src_v.at[b]],
                              rows_v.at[b], sem_r[b]).wait()

    for j in range(DEPTH):
        idx_start(j, j)
    for j in range(DEPTH - 1):
        idx_wait(j, j)
        gather_start(j)

    def body(m, carry):
        for b0 in range(DEPTH):
            k = m * DEPTH + b0
            b = b0
            bn = (b0 + DEPTH - 1) % DEPTH

            @pl.when(k + DEPTH - 1 < CPW)
            def _():
                idx_wait(k + DEPTH - 1, bn)
                gather_start(bn)

            gather_wait(b)
            pltpu.sync_copy(rows_v.at[b], acc.at[dst_v.at[b]], add=True)

            @pl.when(k + DEPTH < CPW)
            def _():
                idx_start(k + DEPTH, b)
        return carry

    lax.fori_loop(0, CPW // DEPTH, body, 0)
    plsc.subcore_barrier()

    pltpu.sync_copy(acc.at[pl.ds(s * OROWS_PT, OROWS_PT)],
                    out_hbm.at[c, pl.ds(s * OROWS_PT, OROWS_PT)])

    @pl.when(s == 0)
    def _():
        pltpu.sync_copy(acc.at[pl.ds(OROWS_PT * NS, OTAIL)],
                        out_hbm.at[c, pl.ds(OROWS_PT * NS, OTAIL)])


def _add_body(a_ref, b_ref, o_ref):
    o_ref[...] = a_ref[...] + b_ref[...]


_BLK = 1000


def _combine(p0, p1):
    return pl.pallas_call(
        _add_body,
        grid=(N_UP // _BLK,),
        in_specs=[pl.BlockSpec((_BLK, D), lambda i: (i, 0))] * 2,
        out_specs=pl.BlockSpec((_BLK, D), lambda i: (i, 0)),
        out_shape=jax.ShapeDtypeStruct((N_UP, D), jnp.float32),
    )(p0, p1)


def kernel(down_nf, edge_index):
    src = edge_index[0]
    dst = edge_index[1]
    pad = E_PAD - E
    src_p = jnp.concatenate([src, jnp.zeros((pad,), jnp.int32)])
    dst_p = jnp.concatenate([dst, jnp.full((pad,), N_UP, jnp.int32)])
    partials = _sc_agg(down_nf, src_p, dst_p)
    return _combine(partials[0], partials[1])


# speedup vs baseline: 2.9879x; 2.7573x over previous
import functools

import jax
import jax.numpy as jnp
from jax import lax
from jax.experimental import pallas as pl
from jax.experimental.pallas import tpu as pltpu
from jax.experimental.pallas import tpu_sc as plsc

N_DOWN = 10000
N_UP = 10000
E = 320000
D = 128

NC = 2
NS = 16
NW = NC * NS

CHUNK = 64
CPW = 160
DEPTH = 5
EPW = CPW * CHUNK
CPW_LAST = (E - (NW - 1) * EPW) // CHUNK
P_ROWS = 10240
ZROWS_PT = P_ROWS // NS
OROWS_PT = 624
OTAIL = N_UP - OROWS_PT * NS

_mesh = plsc.VectorSubcoreMesh(core_axis_name="c", subcore_axis_name="s")


@functools.partial(
    pl.kernel,
    mesh=_mesh,
    out_type=jax.ShapeDtypeStruct((NC, N_UP, D), jnp.float32),
    scratch_types=[
        pltpu.VMEM_SHARED((P_ROWS, D), jnp.float32),
        pltpu.VMEM((DEPTH, CHUNK), jnp.int32),
        pltpu.VMEM((DEPTH, CHUNK), jnp.int32),
        pltpu.VMEM((DEPTH, CHUNK, D), jnp.float32),
        pltpu.VMEM((16, D), jnp.float32),
    ] + [pltpu.SemaphoreType.DMA] * (2 * DEPTH),
)
def _sc_agg(down_hbm, src_hbm, dst_hbm, out_hbm,
            acc, src_v, dst_v, rows_v, zbuf, *sems):
    c = lax.axis_index("c")
    s = lax.axis_index("s")
    wid = c * NS + s
    sem_r = sems[:DEPTH]
    sem_i = sems[DEPTH:]
    base = wid * EPW
    nch = jnp.where(wid == NW - 1, CPW_LAST, CPW)

    for i in range(16):
        for j in range(D // 16):
            zbuf[i, pl.ds(j * 16, 16)] = jnp.zeros((16,), jnp.float32)

    def zero_body(k, carry):
        pltpu.sync_copy(zbuf, acc.at[pl.ds(s * ZROWS_PT + k * 16, 16)])
        return carry

    lax.fori_loop(0, ZROWS_PT // 16, zero_body, 0)
    plsc.subcore_barrier()

    def idx_start(k, b):
        pltpu.async_copy(src_hbm.at[pl.ds(base + k * CHUNK, CHUNK)],
                         src_v.at[b], sem_i[b])
        pltpu.async_copy(dst_hbm.at[pl.ds(base + k * CHUNK, CHUNK)],
                         dst_v.at[b], sem_i[b])

    def idx_wait(k, b):
        pltpu.make_async_copy(src_hbm.at[pl.ds(base + k * CHUNK, CHUNK)],
                              src_v.at[b], sem_i[b]).wait()
        pltpu.make_async_copy(dst_hbm.at[pl.ds(base + k * CHUNK, CHUNK)],
                              dst_v.at[b], sem_i[b]).wait()

    def gather_start(b):
        pltpu.async_copy(down_hbm.at[src_v.at[b]], rows_v.at[b], sem_r[b])

    def gather_wait(b):
        pltpu.make_async_copy(down_hbm.at[src_v.at[b]],
                              rows_v.at[b], sem_r[b]).wait()

    for j in range(DEPTH):
        idx_start(j, j)
    for j in range(DEPTH - 1):
        idx_wait(j, j)
        gather_start(j)

    def body(m, carry):
        for b0 in range(DEPTH):
            k = m * DEPTH + b0
            b = b0
            bn = (b0 + DEPTH - 1) % DEPTH

            @pl.when(k + DEPTH - 1 < nch)
            def _():
                idx_wait(k + DEPTH - 1, bn)
                gather_start(bn)

            gather_wait(b)
            pltpu.sync_copy(rows_v.at[b], acc.at[dst_v.at[b]], add=True)

            @pl.when(k + DEPTH < nch)
            def _():
                idx_start(k + DEPTH, b)
        return carry

    lax.fori_loop(0, nch // DEPTH, body, 0)
    plsc.subcore_barrier()

    pltpu.sync_copy(acc.at[pl.ds(s * OROWS_PT, OROWS_PT)],
                    out_hbm.at[c, pl.ds(s * OROWS_PT, OROWS_PT)])

    @pl.when(s == 0)
    def _():
        pltpu.sync_copy(acc.at[pl.ds(OROWS_PT * NS, OTAIL)],
                        out_hbm.at[c, pl.ds(OROWS_PT * NS, OTAIL)])


def _add_body(a_ref, b_ref, o_ref):
    o_ref[...] = a_ref[...] + b_ref[...]


_BLK = 1000


def _combine(p0, p1):
    return pl.pallas_call(
        _add_body,
        grid=(N_UP // _BLK,),
        in_specs=[pl.BlockSpec((_BLK, D), lambda i: (i, 0))] * 2,
        out_specs=pl.BlockSpec((_BLK, D), lambda i: (i, 0)),
        out_shape=jax.ShapeDtypeStruct((N_UP, D), jnp.float32),
    )(p0, p1)


def kernel(down_nf, edge_index):
    partials = _sc_agg(down_nf, edge_index[0], edge_index[1])
    return _combine(partials[0], partials[1])


# speedup vs baseline: 3.4773x; 1.1638x over previous
import functools

import jax
import jax.numpy as jnp
from jax import lax
from jax.experimental import pallas as pl
from jax.experimental.pallas import tpu as pltpu
from jax.experimental.pallas import tpu_sc as plsc

N_DOWN = 10000
N_UP = 10000
E = 320000
D = 128

NC = 2
NS = 16
NW = NC * NS

CHUNK = 128
CPW = 80
DEPTH = 2
EPW = CPW * CHUNK
CPW_LAST = (E - (NW - 1) * EPW) // CHUNK
P_ROWS = 10240
ZROWS_PT = P_ROWS // NS
OROWS_PT = 624
OTAIL = N_UP - OROWS_PT * NS

_mesh = plsc.VectorSubcoreMesh(core_axis_name="c", subcore_axis_name="s")


@functools.partial(
    pl.kernel,
    mesh=_mesh,
    out_type=jax.ShapeDtypeStruct((NC, N_UP, D), jnp.float32),
    scratch_types=[
        pltpu.VMEM_SHARED((P_ROWS, D), jnp.float32),
        pltpu.VMEM((DEPTH, CHUNK), jnp.int32),
        pltpu.VMEM((DEPTH, CHUNK), jnp.int32),
        pltpu.VMEM((DEPTH, CHUNK, D), jnp.float32),
        pltpu.VMEM((16, D), jnp.float32),
    ] + [pltpu.SemaphoreType.DMA] * (2 * DEPTH),
)
def _sc_agg(down_hbm, src_hbm, dst_hbm, out_hbm,
            acc, src_v, dst_v, rows_v, zbuf, *sems):
    c = lax.axis_index("c")
    s = lax.axis_index("s")
    wid = c * NS + s
    sem_r = sems[:DEPTH]
    sem_i = sems[DEPTH:]
    base = wid * EPW
    nch = jnp.where(wid == NW - 1, CPW_LAST, CPW)

    for i in range(16):
        for j in range(D // 16):
            zbuf[i, pl.ds(j * 16, 16)] = jnp.zeros((16,), jnp.float32)

    def zero_body(k, carry):
        pltpu.sync_copy(zbuf, acc.at[pl.ds(s * ZROWS_PT + k * 16, 16)])
        return carry

    lax.fori_loop(0, ZROWS_PT // 16, zero_body, 0)
    plsc.subcore_barrier()

    def idx_start(k, b):
        pltpu.async_copy(src_hbm.at[pl.ds(base + k * CHUNK, CHUNK)],
                         src_v.at[b], sem_i[b])
        pltpu.async_copy(dst_hbm.at[pl.ds(base + k * CHUNK, CHUNK)],
                         dst_v.at[b], sem_i[b])

    def idx_wait(k, b):
        pltpu.make_async_copy(src_hbm.at[pl.ds(base + k * CHUNK, CHUNK)],
                              src_v.at[b], sem_i[b]).wait()
        pltpu.make_async_copy(dst_hbm.at[pl.ds(base + k * CHUNK, CHUNK)],
                              dst_v.at[b], sem_i[b]).wait()

    def gather_start(b):
        pltpu.async_copy(down_hbm.at[src_v.at[b]], rows_v.at[b], sem_r[b])

    def gather_wait(b):
        pltpu.make_async_copy(down_hbm.at[src_v.at[b]],
                              rows_v.at[b], sem_r[b]).wait()

    for j in range(DEPTH):
        idx_start(j, j)
    for j in range(DEPTH - 1):
        idx_wait(j, j)
        gather_start(j)

    def body(m, carry):
        for b0 in range(DEPTH):
            k = m * DEPTH + b0
            b = b0
            bn = (b0 + DEPTH - 1) % DEPTH

            @pl.when(k + DEPTH - 1 < nch)
            def _():
                idx_wait(k + DEPTH - 1, bn)
                gather_start(bn)

            gather_wait(b)
            pltpu.sync_copy(rows_v.at[b], acc.at[dst_v.at[b]], add=True)

            @pl.when(k + DEPTH < nch)
            def _():
                idx_start(k + DEPTH, b)
        return carry

    lax.fori_loop(0, nch // DEPTH, body, 0)
    plsc.subcore_barrier()

    pltpu.sync_copy(acc.at[pl.ds(s * OROWS_PT, OROWS_PT)],
                    out_hbm.at[c, pl.ds(s * OROWS_PT, OROWS_PT)])

    @pl.when(s == 0)
    def _():
        pltpu.sync_copy(acc.at[pl.ds(OROWS_PT * NS, OTAIL)],
                        out_hbm.at[c, pl.ds(OROWS_PT * NS, OTAIL)])


def _add_body(a_ref, b_ref, o_ref):
    o_ref[...] = a_ref[...] + b_ref[...]


_BLK = 1000


def _combine(p0, p1):
    return pl.pallas_call(
        _add_body,
        grid=(N_UP // _BLK,),
        in_specs=[pl.BlockSpec((_BLK, D), lambda i: (i, 0))] * 2,
        out_specs=pl.BlockSpec((_BLK, D), lambda i: (i, 0)),
        out_shape=jax.ShapeDtypeStruct((N_UP, D), jnp.float32),
    )(p0, p1)


def kernel(down_nf, edge_index):
    partials = _sc_agg(down_nf, edge_index[0], edge_index[1])
    return _combine(partials[0], partials[1])


# speedup vs baseline: 3.5156x; 1.0110x over previous
import functools

import jax
import jax.numpy as jnp
from jax import lax
from jax.experimental import pallas as pl
from jax.experimental.pallas import tpu as pltpu
from jax.experimental.pallas import tpu_sc as plsc

N_DOWN = 10000
N_UP = 10000
E = 320000
D = 128

NC = 2
NS = 16
NW = NC * NS

CHUNK = 128
CPW = 80
DEPTH = 2
EPW = CPW * CHUNK
CPW_LAST = (E - (NW - 1) * EPW) // CHUNK
P_ROWS = 10240
ZROWS_PT = P_ROWS // NS
OROWS_PT = 624
OTAIL = N_UP - OROWS_PT * NS

_mesh = plsc.VectorSubcoreMesh(core_axis_name="c", subcore_axis_name="s")


@functools.partial(
    pl.kernel,
    mesh=_mesh,
    out_type=jax.ShapeDtypeStruct((NC, N_UP, D), jnp.float32),
    scratch_types=[
        pltpu.VMEM_SHARED((P_ROWS, D), jnp.float32),
        pltpu.VMEM((DEPTH, CHUNK), jnp.int32),
        pltpu.VMEM((DEPTH, CHUNK), jnp.int32),
        pltpu.VMEM((DEPTH, CHUNK, D), jnp.float32),
        pltpu.VMEM((32, D), jnp.float32),
    ] + [pltpu.SemaphoreType.DMA] * (2 * DEPTH),
)
def _sc_agg(down_hbm, src_hbm, dst_hbm, out_hbm,
            acc, src_v, dst_v, rows_v, zbuf, *sems):
    c = lax.axis_index("c")
    s = lax.axis_index("s")
    wid = c * NS + s
    sem_r = sems[:DEPTH]
    sem_i = sems[DEPTH:]
    base = wid * EPW
    nch = jnp.where(wid == NW - 1, CPW_LAST, CPW)

    for i in range(32):
        for j in range(D // 16):
            zbuf[i, pl.ds(j * 16, 16)] = jnp.zeros((16,), jnp.float32)

    def zero_body(k, carry):
        pltpu.async_copy(zbuf, acc.at[pl.ds(s * ZROWS_PT + k * 32, 32)],
                         sem_r[0])
        return carry

    lax.fori_loop(0, ZROWS_PT // 32, zero_body, 0)

    def zero_drain(k, carry):
        pltpu.make_async_copy(
            zbuf, acc.at[pl.ds(s * ZROWS_PT + k * 32, 32)],
            sem_r[0]).wait()
        return carry

    lax.fori_loop(0, ZROWS_PT // 32, zero_drain, 0)
    plsc.subcore_barrier()

    def idx_start(k, b):
        pltpu.async_copy(src_hbm.at[pl.ds(base + k * CHUNK, CHUNK)],
                         src_v.at[b], sem_i[b])
        pltpu.async_copy(dst_hbm.at[pl.ds(base + k * CHUNK, CHUNK)],
                         dst_v.at[b], sem_i[b])

    def idx_wait(k, b):
        pltpu.make_async_copy(src_hbm.at[pl.ds(base + k * CHUNK, CHUNK)],
                              src_v.at[b], sem_i[b]).wait()
        pltpu.make_async_copy(dst_hbm.at[pl.ds(base + k * CHUNK, CHUNK)],
                              dst_v.at[b], sem_i[b]).wait()

    def gather_start(b):
        pltpu.async_copy(down_hbm.at[src_v.at[b]], rows_v.at[b], sem_r[b])

    def gather_wait(b):
        pltpu.make_async_copy(down_hbm.at[src_v.at[b]],
                              rows_v.at[b], sem_r[b]).wait()

    for j in range(DEPTH):
        idx_start(j, j)
    for j in range(DEPTH - 1):
        idx_wait(j, j)
        gather_start(j)

    def body(m, carry):
        for b0 in range(DEPTH):
            k = m * DEPTH + b0
            b = b0
            bn = (b0 + DEPTH - 1) % DEPTH

            @pl.when(k + DEPTH - 1 < nch)
            def _():
                idx_wait(k + DEPTH - 1, bn)
                gather_start(bn)

            gather_wait(b)
            pltpu.sync_copy(rows_v.at[b], acc.at[dst_v.at[b]], add=True)

            @pl.when(k + DEPTH < nch)
            def _():
                idx_start(k + DEPTH, b)
        return carry

    lax.fori_loop(0, nch // DEPTH, body, 0)
    plsc.subcore_barrier()

    pltpu.sync_copy(acc.at[pl.ds(s * OROWS_PT, OROWS_PT)],
                    out_hbm.at[c, pl.ds(s * OROWS_PT, OROWS_PT)])

    @pl.when(s == 0)
    def _():
        pltpu.sync_copy(acc.at[pl.ds(OROWS_PT * NS, OTAIL)],
                        out_hbm.at[c, pl.ds(OROWS_PT * NS, OTAIL)])


def _add_body(a_ref, b_ref, o_ref):
    o_ref[...] = a_ref[...] + b_ref[...]


_BLK = 1000


def _combine(p0, p1):
    return pl.pallas_call(
        _add_body,
        grid=(N_UP // _BLK,),
        in_specs=[pl.BlockSpec((_BLK, D), lambda i: (i, 0))] * 2,
        out_specs=pl.BlockSpec((_BLK, D), lambda i: (i, 0)),
        out_shape=jax.ShapeDtypeStruct((N_UP, D), jnp.float32),
    )(p0, p1)


def kernel(down_nf, edge_index):
    partials = _sc_agg(down_nf, edge_index[0], edge_index[1])
    return _combine(partials[0], partials[1])
